# baseline (speedup 1.0000x reference)
import functools

import jax
import jax.numpy as jnp
from jax import lax
from jax.experimental import pallas as pl
from jax.experimental.pallas import tpu as pltpu
from jax.experimental.pallas import tpu_sc as plsc

B = 8
N_SITES = 2048
N_JUNC = 1024
N_TX = 64
J_PER_TX = 16
D = 256
IN_CH = 2 * D
HID = 512

NSPLIT = 2
GB = B // NSPLIT
SLICE_ROWS = 2 * GB * N_JUNC
NW = 32
ROWS_PER_W = SLICE_ROWS // NW
CHUNK = 128
N_CHUNKS = ROWS_PER_W // CHUNK
OROW = 72


def _sc_gather(table, idx):
    mesh = plsc.VectorSubcoreMesh(core_axis_name="c", subcore_axis_name="s")

    @functools.partial(
        pl.kernel,
        mesh=mesh,
        out_type=jax.ShapeDtypeStruct((SLICE_ROWS, D), jnp.float32),
        scratch_types=[
            pltpu.VMEM((N_CHUNKS, CHUNK), jnp.int32),
            pltpu.VMEM((CHUNK, D), jnp.float32),
            pltpu.VMEM((CHUNK, D), jnp.float32),
            pltpu.SemaphoreType.DMA,
            pltpu.SemaphoreType.DMA,
        ],
    )
    def k(table_hbm, idx_hbm, out_hbm, idx_v, buf0, buf1, sem0, sem1):
        wid = lax.axis_index("s") * 2 + lax.axis_index("c")
        base = wid * ROWS_PER_W
        pltpu.sync_copy(idx_hbm.at[wid], idx_v)
        bufs = (buf0, buf1)
        sems = (sem0, sem1)
        prev = pltpu.async_copy(table_hbm.at[idx_v.at[0]], bufs[0], sems[0])
        for c in range(1, N_CHUNKS):
            cur = pltpu.async_copy(table_hbm.at[idx_v.at[c]], bufs[c % 2], sems[c % 2])
            prev.wait()
            pltpu.sync_copy(bufs[(c - 1) % 2],
                            out_hbm.at[pl.ds(base + (c - 1) * CHUNK, CHUNK)])
            prev = cur
        prev.wait()
        pltpu.sync_copy(bufs[(N_CHUNKS - 1) % 2],
                        out_hbm.at[pl.ds(base + (N_CHUNKS - 1) * CHUNK, CHUNK)])

    return k(table, idx)


def _tc_body(xd_ref, xa_ref, w1d_ref, w1a_ref, b1_ref, w2_ref, b2_ref,
             w3_ref, ids_ref, scal_ref, out_ref):
    xd = xd_ref[...].astype(jnp.bfloat16)
    xa = xa_ref[...].astype(jnp.bfloat16)
    h1 = (jnp.dot(xd, w1d_ref[...], preferred_element_type=jnp.float32)
          + jnp.dot(xa, w1a_ref[...], preferred_element_type=jnp.float32))
    h1 = jnp.maximum(h1 + b1_ref[0:1], 0.0).astype(jnp.bfloat16)
    h2 = jnp.dot(h1, w2_ref[...], preferred_element_type=jnp.float32)
    h2 = jnp.maximum(h2 + b2_ref[0:1], 0.0).astype(jnp.bfloat16)

    ids = ids_ref[0]
    jidx = lax.broadcasted_iota(jnp.int32, (N_TX, N_JUNC), 1)
    a = jnp.zeros((N_TX, N_JUNC), jnp.float32)
    for k in range(J_PER_TX):
        a = a + (ids[:, k:k + 1] == jidx).astype(jnp.float32)
    a = a.astype(jnp.bfloat16)
    th = jnp.dot(a, h2, preferred_element_type=jnp.float32)
    w3b = jnp.broadcast_to(w3_ref[...], (HID, 128)).astype(jnp.bfloat16)
    tp = jnp.dot(th.astype(jnp.bfloat16), w3b,
                 preferred_element_type=jnp.float32)
    tp = tp + scal_ref[0, 0] * J_PER_TX

    rows = lax.broadcasted_iota(jnp.int32, (OROW, 128), 0)
    tp_pad = jnp.concatenate(
        [tp, jnp.zeros((OROW - N_TX, 128), jnp.float32)], axis=0)
    neg_inf = jnp.float32(-jnp.inf)
    v = jnp.where(rows < N_TX, tp_pad,
                  jnp.where(rows == N_TX, scal_ref[0, 1], neg_inf))
    m = jnp.max(v, axis=0, keepdims=True)
    e = jnp.exp(v - m)
    out_ref[0] = e / jnp.sum(e, axis=0, keepdims=True)


def _tc_mlp(embs, w1d, w1a, b1r, w2, b2r, w3, ids, scal, sl):
    return pl.pallas_call(
        _tc_body,
        grid=(GB,),
        in_specs=[
            pl.BlockSpec((N_JUNC, D), lambda s: (s, 0)),
            pl.BlockSpec((N_JUNC, D), lambda s: (GB + s, 0)),
            pl.BlockSpec((D, HID), lambda s: (0, 0)),
            pl.BlockSpec((D, HID), lambda s: (0, 0)),
            pl.BlockSpec((1, HID), lambda s: (0, 0)),
            pl.BlockSpec((HID, HID), lambda s: (0, 0)),
            pl.BlockSpec((1, HID), lambda s: (0, 0)),
            pl.BlockSpec((HID, 1), lambda s: (0, 0)),
            pl.BlockSpec((1, N_TX, 128), lambda s, q=sl: (q * GB + s, 0, 0)),
            pl.BlockSpec((1, 2), lambda s: (0, 0), memory_space=pltpu.SMEM),
        ],
        out_specs=pl.BlockSpec((1, OROW, 128), lambda s: (s, 0, 0)),
        out_shape=jax.ShapeDtypeStruct((GB, OROW, 128), jnp.float32),
    )(embs, embs, w1d, w1a, b1r, w2, b2r, w3, ids, scal)


def kernel(splice_site_reps, junction_indices, transcript_junction_ids,
           W1, b1, W2, b2, W3, b3, ref_potential):
    table = splice_site_reps.reshape(B * N_SITES, D)
    idx = (junction_indices.astype(jnp.int32)
           + (jnp.arange(B, dtype=jnp.int32) * N_SITES)[:, None, None])
    idx = jnp.transpose(idx.reshape(NSPLIT, GB, N_JUNC, 2),
                        (0, 3, 1, 2))

    w1d = W1[:D].astype(jnp.bfloat16)
    w1a = W1[D:].astype(jnp.bfloat16)
    w2 = W2.astype(jnp.bfloat16)
    w3 = W3
    b1r = b1.reshape(1, HID)
    b2r = b2.reshape(1, HID)
    scal = jnp.concatenate([b3, ref_potential]).reshape(1, 2)
    ids = jnp.pad(transcript_junction_ids.astype(jnp.int32),
                  ((0, 0), (0, 0), (0, 128 - J_PER_TX)))

    outs = []
    for sl in range(NSPLIT):
        idx_s = idx[sl].reshape(NW, N_CHUNKS, CHUNK)
        embs = _sc_gather(table, idx_s)
        outs.append(_tc_mlp(embs, w1d, w1a, b1r, w2, b2r, w3, ids, scal, sl))
    out = jnp.concatenate(outs, axis=0)
    return out[:, :N_TX + 1, 0]


# speedup vs baseline: 1.0099x; 1.0099x over previous
import functools

import jax
import jax.numpy as jnp
from jax import lax
from jax.experimental import pallas as pl
from jax.experimental.pallas import tpu as pltpu
from jax.experimental.pallas import tpu_sc as plsc

B = 8
N_SITES = 2048
N_JUNC = 1024
N_TX = 64
J_PER_TX = 16
D = 256
IN_CH = 2 * D
HID = 512

NSPLIT = 2
GB = B // NSPLIT
SLICE_ROWS = 2 * GB * N_JUNC
NW = 32
ROWS_PER_W = SLICE_ROWS // NW
CHUNK = 128
N_CHUNKS = ROWS_PER_W // CHUNK
OROW = 72


def _sc_gather(table, idx):
    mesh = plsc.VectorSubcoreMesh(core_axis_name="c", subcore_axis_name="s")

    @functools.partial(
        pl.kernel,
        mesh=mesh,
        out_type=jax.ShapeDtypeStruct((SLICE_ROWS, D), jnp.float32),
        scratch_types=[
            pltpu.VMEM((N_CHUNKS, CHUNK), jnp.int32),
            pltpu.VMEM((CHUNK, D), jnp.float32),
            pltpu.VMEM((CHUNK, D), jnp.float32),
            pltpu.SemaphoreType.DMA,
            pltpu.SemaphoreType.DMA,
        ],
    )
    def k(table_hbm, idx_hbm, out_hbm, idx_v, buf0, buf1, sem0, sem1):
        wid = lax.axis_index("s") * 2 + lax.axis_index("c")
        base = wid * ROWS_PER_W
        pltpu.sync_copy(idx_hbm.at[wid], idx_v)
        bufs = (buf0, buf1)
        sems = (sem0, sem1)
        prev = pltpu.async_copy(table_hbm.at[idx_v.at[0]], bufs[0], sems[0])
        for c in range(1, N_CHUNKS):
            cur = pltpu.async_copy(table_hbm.at[idx_v.at[c]], bufs[c % 2], sems[c % 2])
            prev.wait()
            pltpu.sync_copy(bufs[(c - 1) % 2],
                            out_hbm.at[pl.ds(base + (c - 1) * CHUNK, CHUNK)])
            prev = cur
        prev.wait()
        pltpu.sync_copy(bufs[(N_CHUNKS - 1) % 2],
                        out_hbm.at[pl.ds(base + (N_CHUNKS - 1) * CHUNK, CHUNK)])

    return k(table, idx)


def _tc_body(xd_ref, xa_ref, w1d_ref, w1a_ref, b1_ref, w2_ref, b2_ref,
             w3_ref, ids_ref, scal_ref, out_ref):
    xd = xd_ref[...].astype(jnp.bfloat16)
    xa = xa_ref[...].astype(jnp.bfloat16)
    h1 = (jnp.dot(xd, w1d_ref[...], preferred_element_type=jnp.float32)
          + jnp.dot(xa, w1a_ref[...], preferred_element_type=jnp.float32))
    h1 = jnp.maximum(h1 + b1_ref[0:1], 0.0).astype(jnp.bfloat16)
    h2 = jnp.dot(h1, w2_ref[...], preferred_element_type=jnp.float32)
    h2 = jnp.maximum(h2 + b2_ref[0:1], 0.0).astype(jnp.bfloat16)

    ids = ids_ref[0]
    jidx = lax.broadcasted_iota(jnp.int32, (N_TX, N_JUNC), 1)
    a = jnp.zeros((N_TX, N_JUNC), jnp.float32)
    for k in range(J_PER_TX):
        a = a + (ids[:, k:k + 1] == jidx).astype(jnp.float32)
    a = a.astype(jnp.bfloat16)
    th = jnp.dot(a, h2, preferred_element_type=jnp.float32)
    tp = jnp.dot(th.astype(jnp.bfloat16), w3_ref[...],
                 preferred_element_type=jnp.float32)
    tp = tp + scal_ref[0, 0] * J_PER_TX

    rows = lax.broadcasted_iota(jnp.int32, (OROW, 128), 0)
    tp_pad = jnp.concatenate(
        [tp, jnp.zeros((OROW - N_TX, 128), jnp.float32)], axis=0)
    neg_inf = jnp.float32(-jnp.inf)
    v = jnp.where(rows < N_TX, tp_pad,
                  jnp.where(rows == N_TX, scal_ref[0, 1], neg_inf))
    m = jnp.max(v, axis=0, keepdims=True)
    e = jnp.exp(v - m)
    out_ref[0] = e / jnp.sum(e, axis=0, keepdims=True)


def _tc_mlp(embs, w1d, w1a, b1r, w2, b2r, w3, ids, scal, sl):
    return pl.pallas_call(
        _tc_body,
        grid=(GB,),
        in_specs=[
            pl.BlockSpec((N_JUNC, D), lambda s: (s, 0)),
            pl.BlockSpec((N_JUNC, D), lambda s: (GB + s, 0)),
            pl.BlockSpec((D, HID), lambda s: (0, 0)),
            pl.BlockSpec((D, HID), lambda s: (0, 0)),
            pl.BlockSpec((1, HID), lambda s: (0, 0)),
            pl.BlockSpec((HID, HID), lambda s: (0, 0)),
            pl.BlockSpec((1, HID), lambda s: (0, 0)),
            pl.BlockSpec((HID, 128), lambda s: (0, 0)),
            pl.BlockSpec((1, N_TX, 128), lambda s, q=sl: (q * GB + s, 0, 0)),
            pl.BlockSpec((1, 2), lambda s: (0, 0), memory_space=pltpu.SMEM),
        ],
        out_specs=pl.BlockSpec((1, OROW, 128), lambda s: (s, 0, 0)),
        out_shape=jax.ShapeDtypeStruct((GB, OROW, 128), jnp.float32),
    )(embs, embs, w1d, w1a, b1r, w2, b2r, w3, ids, scal)


def kernel(splice_site_reps, junction_indices, transcript_junction_ids,
           W1, b1, W2, b2, W3, b3, ref_potential):
    table = splice_site_reps.reshape(B * N_SITES, D)
    idx = (junction_indices.astype(jnp.int32)
           + (jnp.arange(B, dtype=jnp.int32) * N_SITES)[:, None, None])
    idx = jnp.transpose(idx.reshape(NSPLIT, GB, N_JUNC, 2),
                        (0, 3, 1, 2))

    w1d = W1[:D].astype(jnp.bfloat16)
    w1a = W1[D:].astype(jnp.bfloat16)
    w2 = W2.astype(jnp.bfloat16)
    w3 = jnp.tile(W3, (1, 128)).astype(jnp.bfloat16)
    b1r = b1.reshape(1, HID)
    b2r = b2.reshape(1, HID)
    scal = jnp.concatenate([b3, ref_potential]).reshape(1, 2)
    ids = jnp.pad(transcript_junction_ids.astype(jnp.int32),
                  ((0, 0), (0, 0), (0, 128 - J_PER_TX)))

    outs = []
    for sl in range(NSPLIT):
        idx_s = idx[sl].reshape(NW, N_CHUNKS, CHUNK)
        embs = _sc_gather(table, idx_s)
        outs.append(_tc_mlp(embs, w1d, w1a, b1r, w2, b2r, w3, ids, scal, sl))
    out = jnp.concatenate(outs, axis=0)
    return out[:, :N_TX + 1, 0]
